# Initial kernel scaffold; baseline (speedup 1.0000x reference)
import jax
import jax.numpy as jnp
from jax.experimental import pallas as pl


def kernel(pred, gt, heatmap):
    raise NotImplementedError("write your pallas kernel here")


# speedup vs baseline: 1.3923x; 1.3923x over previous
import functools

import jax
import jax.numpy as jnp
from jax import lax
from jax.experimental import pallas as pl
from jax.experimental.pallas import tpu as pltpu
from jax.experimental.pallas import tpu_sc as plsc

_NC = 2
_NS = 16
_L = 16


def _argmax_body(hm_ref, val_ref, idx_ref):
    x = hm_ref[...]
    mx = jnp.max(x, axis=1, keepdims=True)
    ii = lax.broadcasted_iota(jnp.int32, x.shape, 1)
    first = jnp.min(jnp.where(x == mx, ii, x.shape[1]), axis=1)
    val_ref[...] = mx
    idx_ref[...] = first[:, None]


def _tc_argmax(hm2, rows_per_block=8):
    n_rows, hw = hm2.shape
    r = rows_per_block
    return pl.pallas_call(
        _argmax_body,
        grid=(n_rows // r,),
        in_specs=[pl.BlockSpec((r, hw), lambda i: (i, 0))],
        out_specs=[
            pl.BlockSpec((r, 1), lambda i: (i, 0)),
            pl.BlockSpec((r, 1), lambda i: (i, 0)),
        ],
        out_shape=[
            jax.ShapeDtypeStruct((n_rows, 1), jnp.float32),
            jax.ShapeDtypeStruct((n_rows, 1), jnp.int32),
        ],
    )(hm2)


def _sc_loss_body(hw, n_ch, kp, pred_ref, gtt_ref, idx_ref, vals_ref, out_ref,
                  pred_v, gt_v, idx_v, vals_v, out_v):
    b_count = idx_ref.shape[0]
    wid = lax.axis_index("s") * _NC + lax.axis_index("c")

    @pl.when(wid < b_count)
    def _():
        b = wid
        pltpu.sync_copy(pred_ref.at[b, pl.ds(0, n_ch * hw)], pred_v)
        pltpu.sync_copy(gtt_ref.at[b], gt_v)
        pltpu.sync_copy(idx_ref.at[b], idx_v)
        pltpu.sync_copy(vals_ref.at[b], vals_v)
        acc = jnp.zeros((_L,), jnp.float32)
        for kc in range(kp // _L):
            sl = pl.ds(kc * _L, _L)
            i16 = idx_v[sl]
            v16 = vals_v[sl]
            cl = jnp.zeros((_L,), jnp.float32)
            for c in range(n_ch):
                g = plsc.load_gather(pred_v, [i16 + c * hw])
                t = gt_v[pl.ds(c * kp + kc * _L, _L)]
                d = g - t
                cl = cl + d * d
            acc = acc + jnp.where(v16 == 1.0, cl, 0.0)
        tot = jnp.sum(acc)
        out_v[...] = jnp.full((_L,), tot, jnp.float32)
        pltpu.sync_copy(out_v, out_ref.at[b])


def kernel(pred, gt, heatmap):
    B, K, H, W = heatmap.shape
    HW = H * W
    n_ch = 7
    kp = ((K + _L - 1) // _L) * _L

    vals2, idx2 = _tc_argmax(heatmap.reshape(B * K, HW))

    idxp = jnp.pad(idx2.reshape(B, K), ((0, 0), (0, kp - K)))
    valsp = jnp.pad(vals2.reshape(B, K), ((0, 0), (0, kp - K)))
    gtt = jnp.pad(gt[:, :, :n_ch].transpose(0, 2, 1), ((0, 0), (0, 0), (0, kp - K)))

    sck = pl.kernel(
        functools.partial(_sc_loss_body, HW, n_ch, kp),
        out_type=jax.ShapeDtypeStruct((B, _L), jnp.float32),
        mesh=plsc.VectorSubcoreMesh(core_axis_name="c", subcore_axis_name="s"),
        compiler_params=pltpu.CompilerParams(needs_layout_passes=False),
        scratch_types=[
            pltpu.VMEM((n_ch * HW,), jnp.float32),
            pltpu.VMEM((n_ch * kp,), jnp.float32),
            pltpu.VMEM((kp,), jnp.int32),
            pltpu.VMEM((kp,), jnp.float32),
            pltpu.VMEM((_L,), jnp.float32),
        ],
    )
    out2 = sck(pred.reshape(B, pred.shape[1] * HW),
               gtt.reshape(B, n_ch * kp), idxp, valsp)
    return out2[:, 0]


# speedup vs baseline: 2.1638x; 1.5541x over previous
import functools

import jax
import jax.numpy as jnp
from jax import lax
from jax.experimental import pallas as pl
from jax.experimental.pallas import tpu as pltpu
from jax.experimental.pallas import tpu_sc as plsc

_NC = 2
_NS = 16
_L = 16


def _argmax_body(hm_ref, val_ref, idx_ref):
    x = hm_ref[0]
    kb, h, w = x.shape
    m1 = jnp.max(x, axis=1)
    ih = lax.broadcasted_iota(jnp.int32, x.shape, 1)
    fh = jnp.min(jnp.where(x == m1[:, None, :], ih, h), axis=1)
    m2 = jnp.max(m1, axis=1)
    iw = lax.broadcasted_iota(jnp.int32, (kb, w), 1)
    flat = fh * w + iw
    first = jnp.min(jnp.where(m1 == m2[:, None], flat, h * w), axis=1)
    val_ref[0, 0] = m2
    idx_ref[0, 0] = first


def _tc_argmax(hm, kb=10):
    b, k, h, w = hm.shape
    nk = k // kb
    vals, idx = pl.pallas_call(
        _argmax_body,
        grid=(b, nk),
        in_specs=[pl.BlockSpec((1, kb, h, w), lambda i, j: (i, j, 0, 0))],
        out_specs=[
            pl.BlockSpec((1, 1, kb), lambda i, j: (i * nk + j, 0, 0)),
            pl.BlockSpec((1, 1, kb), lambda i, j: (i * nk + j, 0, 0)),
        ],
        out_shape=[
            jax.ShapeDtypeStruct((b * nk, 1, kb), jnp.float32),
            jax.ShapeDtypeStruct((b * nk, 1, kb), jnp.int32),
        ],
    )(hm)
    return vals.reshape(b, k), idx.reshape(b, k)


def _sc_loss_body(hw, n_ch, kp, pred_ref, gtt_ref, idx_ref, vals_ref, out_ref,
                  pred_v, gt_v, idx_v, vals_v, out_v):
    b_count = idx_ref.shape[0]
    wid = lax.axis_index("s") * _NC + lax.axis_index("c")

    @pl.when(wid < b_count)
    def _():
        b = wid
        pltpu.sync_copy(pred_ref.at[b, pl.ds(0, n_ch * hw)], pred_v)
        pltpu.sync_copy(gtt_ref.at[b], gt_v)
        pltpu.sync_copy(idx_ref.at[b], idx_v)
        pltpu.sync_copy(vals_ref.at[b], vals_v)
        acc = jnp.zeros((_L,), jnp.float32)
        for kc in range(kp // _L):
            sl = pl.ds(kc * _L, _L)
            i16 = idx_v[sl]
            v16 = vals_v[sl]
            cl = jnp.zeros((_L,), jnp.float32)
            for c in range(n_ch):
                g = plsc.load_gather(pred_v, [i16 + c * hw])
                t = gt_v[pl.ds(c * kp + kc * _L, _L)]
                d = g - t
                cl = cl + d * d
            acc = acc + jnp.where(v16 == 1.0, cl, 0.0)
        tot = jnp.sum(acc)
        out_v[...] = jnp.full((_L,), tot, jnp.float32)
        pltpu.sync_copy(out_v, out_ref.at[b])


def kernel(pred, gt, heatmap):
    B, K, H, W = heatmap.shape
    HW = H * W
    n_ch = 7
    kp = ((K + _L - 1) // _L) * _L

    vals2, idx2 = _tc_argmax(heatmap)

    idxp = jnp.pad(idx2, ((0, 0), (0, kp - K)))
    valsp = jnp.pad(vals2, ((0, 0), (0, kp - K)))
    gtt = jnp.pad(gt[:, :, :n_ch].transpose(0, 2, 1), ((0, 0), (0, 0), (0, kp - K)))

    sck = pl.kernel(
        functools.partial(_sc_loss_body, HW, n_ch, kp),
        out_type=jax.ShapeDtypeStruct((B, _L), jnp.float32),
        mesh=plsc.VectorSubcoreMesh(core_axis_name="c", subcore_axis_name="s"),
        compiler_params=pltpu.CompilerParams(needs_layout_passes=False),
        scratch_types=[
            pltpu.VMEM((n_ch * HW,), jnp.float32),
            pltpu.VMEM((n_ch * kp,), jnp.float32),
            pltpu.VMEM((kp,), jnp.int32),
            pltpu.VMEM((kp,), jnp.float32),
            pltpu.VMEM((_L,), jnp.float32),
        ],
    )
    out2 = sck(pred.reshape(B, pred.shape[1] * HW),
               gtt.reshape(B, n_ch * kp), idxp, valsp)
    return out2[:, 0]


# speedup vs baseline: 2.6964x; 1.2461x over previous
import functools

import jax
import jax.numpy as jnp
from jax import lax
from jax.experimental import pallas as pl
from jax.experimental.pallas import tpu as pltpu
from jax.experimental.pallas import tpu_sc as plsc

_NC = 2
_NS = 16
_L = 16


def _argmax_body(k_total, hm_ref, val_ref, idx_ref):
    x = hm_ref[0]
    kb, h, w = x.shape
    j = pl.program_id(1)
    m1 = jnp.max(x, axis=1)
    m2 = jnp.max(m1, axis=1)
    ih = lax.broadcasted_iota(jnp.int32, (1, h, w), 1)
    iw = lax.broadcasted_iota(jnp.int32, (1, h, w), 2)
    flatf = (ih * w + iw).astype(jnp.float32)
    cand = jnp.where(x == m2[:, None, None], flatf, float(h * w))
    first = jnp.min(cand, axis=(1, 2)).astype(jnp.int32)
    kid = j * kb + lax.broadcasted_iota(jnp.int32, (kb,), 0)
    valid = kid < k_total
    val_ref[0, 0] = jnp.where(valid, m2, 0.0)
    idx_ref[0, 0] = jnp.where(valid, first, 0)


def _tc_argmax(hm, kp, kb=16):
    b, k, h, w = hm.shape
    nk = kp // kb
    vals, idx = pl.pallas_call(
        functools.partial(_argmax_body, k),
        grid=(b, nk),
        in_specs=[pl.BlockSpec((1, kb, h, w), lambda i, j: (i, j, 0, 0))],
        out_specs=[
            pl.BlockSpec((1, 1, kb), lambda i, j: (i * nk + j, 0, 0)),
            pl.BlockSpec((1, 1, kb), lambda i, j: (i * nk + j, 0, 0)),
        ],
        out_shape=[
            jax.ShapeDtypeStruct((b * nk, 1, kb), jnp.float32),
            jax.ShapeDtypeStruct((b * nk, 1, kb), jnp.int32),
        ],
    )(hm)
    return vals.reshape(b, kp), idx.reshape(b, kp)


def _sc_loss_body(n_ch, kp, pred_ref, gt_ref, idx_ref, vals_ref, out_ref,
                  pred_v, gt_v, idx_v, vals_v, out_v):
    b_count, k_real = gt_ref.shape[0], gt_ref.shape[1]
    w = pred_ref.shape[3]
    shift = w.bit_length() - 1
    wid = lax.axis_index("s") * _NC + lax.axis_index("c")

    @pl.when(wid < b_count)
    def _():
        b = wid
        pltpu.sync_copy(pred_ref.at[b, pl.ds(0, n_ch)], pred_v)
        pltpu.sync_copy(gt_ref.at[b], gt_v.at[pl.ds(0, k_real)])
        pltpu.sync_copy(idx_ref.at[b], idx_v)
        pltpu.sync_copy(vals_ref.at[b], vals_v)
        lanes = lax.broadcasted_iota(jnp.int32, (_L,), 0)
        acc = jnp.zeros((_L,), jnp.float32)
        for kc in range(kp // _L):
            sl = pl.ds(kc * _L, _L)
            i16 = idx_v[sl]
            v16 = vals_v[sl]
            x16 = lax.shift_right_logical(i16, shift)
            y16 = lax.bitwise_and(i16, w - 1)
            kk = kc * _L + lanes
            cl = jnp.zeros((_L,), jnp.float32)
            for c in range(n_ch):
                cvec = jnp.full((_L,), c, jnp.int32)
                g = plsc.load_gather(pred_v, [cvec, x16, y16])
                t = plsc.load_gather(gt_v, [kk, cvec])
                d = g - t
                cl = cl + d * d
            acc = acc + jnp.where(v16 == 1.0, cl, 0.0)
        tot = jnp.sum(acc)
        out_v[...] = jnp.full((_L,), tot, jnp.float32)
        pltpu.sync_copy(out_v, out_ref.at[b])


def kernel(pred, gt, heatmap):
    B, K, H, W = heatmap.shape
    HW = H * W
    n_ch = 7
    kp = ((K + _L - 1) // _L) * _L

    valsp, idxp = _tc_argmax(heatmap, kp)

    sck = pl.kernel(
        functools.partial(_sc_loss_body, n_ch, kp),
        out_type=jax.ShapeDtypeStruct((B, _L), jnp.float32),
        mesh=plsc.VectorSubcoreMesh(core_axis_name="c", subcore_axis_name="s"),
        compiler_params=pltpu.CompilerParams(needs_layout_passes=False),
        scratch_types=[
            pltpu.VMEM((n_ch, H, W), jnp.float32),
            pltpu.VMEM((kp, gt.shape[2]), jnp.float32),
            pltpu.VMEM((kp,), jnp.int32),
            pltpu.VMEM((kp,), jnp.float32),
            pltpu.VMEM((_L,), jnp.float32),
        ],
    )
    out2 = sck(pred, gt, idxp, valsp)
    return out2[:, 0]
